# Initial kernel scaffold; baseline (speedup 1.0000x reference)
import jax
import jax.numpy as jnp
from jax.experimental import pallas as pl


def kernel(ids_or_weights, embedding_weight):
    raise NotImplementedError("write your pallas kernel here")


# speedup vs baseline: 6.2005x; 6.2005x over previous
import functools
import math

import jax
import jax.numpy as jnp
from jax import lax
from jax.experimental import pallas as pl
from jax.experimental.pallas import tpu as pltpu
from jax.experimental.pallas import tpu_sc as plsc

NUM_CLASSES = 100000
EMBED_DIM = 32
SQRT_D = math.sqrt(EMBED_DIM)

NC = 2
NS = 16
NW = NC * NS


_NORM_BLOCK = 2000


def _normalize_body(t_ref, o_ref):
    x = t_ref[...]
    ssq = jnp.sum(x * x, axis=-1, keepdims=True)
    norm = jnp.maximum(jnp.sqrt(ssq), 1e-12)
    o_ref[...] = x * (SQRT_D / norm)


def _normalize_table(table):
    n = table.shape[0]
    grid = n // _NORM_BLOCK
    return pl.pallas_call(
        _normalize_body,
        out_shape=jax.ShapeDtypeStruct(table.shape, table.dtype),
        grid=(grid,),
        in_specs=[pl.BlockSpec((_NORM_BLOCK, EMBED_DIM), lambda i: (i, 0))],
        out_specs=pl.BlockSpec((_NORM_BLOCK, EMBED_DIM), lambda i: (i, 0)),
    )(table)


_CHUNK = 2048


def _make_gather(total_rows):
    b_per_w = total_rows // NW
    n_chunks = b_per_w // _CHUNK
    mesh = plsc.VectorSubcoreMesh(
        core_axis_name="c", subcore_axis_name="s", num_cores=NC, num_subcores=NS
    )

    @functools.partial(
        pl.kernel,
        out_type=jax.ShapeDtypeStruct((total_rows, EMBED_DIM), jnp.float32),
        mesh=mesh,
        scratch_types=[
            pltpu.VMEM((_CHUNK,), jnp.int32),
            pltpu.VMEM((_CHUNK, EMBED_DIM), jnp.float32),
            pltpu.SemaphoreType.DMA,
        ],
        compiler_params=pltpu.CompilerParams(use_tc_tiling_on_sc=False),
    )
    def gather_k(idx_hbm, table_hbm, out_hbm, idx_v, rows_v, sem):
        wid = lax.axis_index("s") * NC + lax.axis_index("c")
        base0 = wid * b_per_w

        def step(c, carry):
            base = pl.multiple_of(base0 + c * _CHUNK, _CHUNK)
            pltpu.sync_copy(idx_hbm.at[pl.ds(base, _CHUNK)], idx_v)
            pltpu.async_copy(table_hbm.at[idx_v], rows_v, sem).wait()
            pltpu.sync_copy(rows_v, out_hbm.at[pl.ds(base, _CHUNK)])
            return carry

        lax.fori_loop(0, n_chunks, step, 0)

    return gather_k


def kernel(ids_or_weights, embedding_weight):
    table_n = _normalize_table(embedding_weight)
    batch, hist = ids_or_weights.shape
    idx = ids_or_weights.reshape(-1)
    out = _make_gather(batch * hist)(idx, table_n)
    return out.reshape(batch, hist, EMBED_DIM)


# speedup vs baseline: 6.3288x; 1.0207x over previous
import functools
import math

import jax
import jax.numpy as jnp
from jax import lax
from jax.experimental import pallas as pl
from jax.experimental.pallas import tpu as pltpu
from jax.experimental.pallas import tpu_sc as plsc

NUM_CLASSES = 100000
EMBED_DIM = 32
SQRT_D = math.sqrt(EMBED_DIM)

NC = 2
NS = 16
NW = NC * NS


_NORM_BLOCK = 2000


def _normalize_body(t_ref, o_ref):
    x = t_ref[...]
    ssq = jnp.sum(x * x, axis=-1, keepdims=True)
    norm = jnp.maximum(jnp.sqrt(ssq), 1e-12)
    o_ref[...] = x * (SQRT_D / norm)


def _normalize_table(table):
    n = table.shape[0]
    grid = n // _NORM_BLOCK
    return pl.pallas_call(
        _normalize_body,
        out_shape=jax.ShapeDtypeStruct(table.shape, table.dtype),
        grid=(grid,),
        in_specs=[pl.BlockSpec((_NORM_BLOCK, EMBED_DIM), lambda i: (i, 0))],
        out_specs=pl.BlockSpec((_NORM_BLOCK, EMBED_DIM), lambda i: (i, 0)),
    )(table)


_CHUNK = 1600


def _make_gather(total_rows):
    b_per_w = total_rows // NW
    n_chunks = b_per_w // _CHUNK
    assert n_chunks >= 6 and n_chunks % 2 == 0
    mesh = plsc.VectorSubcoreMesh(
        core_axis_name="c", subcore_axis_name="s", num_cores=NC, num_subcores=NS
    )

    @functools.partial(
        pl.kernel,
        out_type=jax.ShapeDtypeStruct((total_rows, EMBED_DIM), jnp.float32),
        mesh=mesh,
        scratch_types=[
            pltpu.VMEM((_CHUNK,), jnp.int32),
            pltpu.VMEM((_CHUNK,), jnp.int32),
            pltpu.VMEM((_CHUNK, EMBED_DIM), jnp.float32),
            pltpu.VMEM((_CHUNK, EMBED_DIM), jnp.float32),
            pltpu.SemaphoreType.DMA,
            pltpu.SemaphoreType.DMA,
            pltpu.SemaphoreType.DMA,
            pltpu.SemaphoreType.DMA,
            pltpu.SemaphoreType.DMA,
            pltpu.SemaphoreType.DMA,
        ],
        compiler_params=pltpu.CompilerParams(use_tc_tiling_on_sc=False),
    )
    def gather_k(idx_hbm, table_hbm, out_hbm, i0, i1, r0, r1, si0, si1, sg0, sg1, so0, so1):
        wid = lax.axis_index("s") * NC + lax.axis_index("c")
        base0 = wid * b_per_w
        I, R = (i0, i1), (r0, r1)
        SI, SG, SO = (si0, si1), (sg0, sg1), (so0, so1)

        def off(c):
            return pl.multiple_of(base0 + c * _CHUNK, 8)

        def idx_start(c, b):
            pltpu.async_copy(idx_hbm.at[pl.ds(off(c), _CHUNK)], I[b], SI[b])

        def idx_wait(b):
            pltpu.make_async_copy(
                idx_hbm.at[pl.ds(off(0), _CHUNK)], I[b], SI[b]
            ).wait()

        def gather_start(b):
            pltpu.async_copy(table_hbm.at[I[b]], R[b], SG[b])

        def gather_wait(b):
            pltpu.make_async_copy(table_hbm.at[I[b]], R[b], SG[b]).wait()

        def out_start(c, b):
            pltpu.async_copy(R[b], out_hbm.at[pl.ds(off(c), _CHUNK)], SO[b])

        def out_wait(b):
            pltpu.make_async_copy(
                R[b], out_hbm.at[pl.ds(off(0), _CHUNK)], SO[b]
            ).wait()

        def step(c, b, prefetch=True, first=False):
            gather_wait(1 - b)
            out_start(c - 1, 1 - b)
            if prefetch:
                idx_start(c + 1, 1 - b)
            idx_wait(b)
            if not first:
                out_wait(b)
            gather_start(b)

        idx_start(0, 0)
        idx_wait(0)
        gather_start(0)
        idx_start(1, 1)
        step(1, 1, prefetch=True, first=True)

        def body(g, carry):
            c = 2 * g + 2
            step(c, 0)
            step(c + 1, 1)
            return carry

        lax.fori_loop(0, (n_chunks - 4) // 2, body, 0)

        step(n_chunks - 2, (n_chunks - 2) % 2, prefetch=True)
        step(n_chunks - 1, (n_chunks - 1) % 2, prefetch=False)
        bl = (n_chunks - 1) % 2
        gather_wait(bl)
        out_start(n_chunks - 1, bl)
        out_wait(1 - bl)
        out_wait(bl)

    return gather_k


def kernel(ids_or_weights, embedding_weight):
    table_n = _normalize_table(embedding_weight)
    batch, hist = ids_or_weights.shape
    idx = ids_or_weights.reshape(-1)
    out = _make_gather(batch * hist)(idx, table_n)
    return out.reshape(batch, hist, EMBED_DIM)


# speedup vs baseline: 6.3366x; 1.0012x over previous
import functools
import math

import jax
import jax.numpy as jnp
from jax import lax
from jax.experimental import pallas as pl
from jax.experimental.pallas import tpu as pltpu
from jax.experimental.pallas import tpu_sc as plsc

NUM_CLASSES = 100000
EMBED_DIM = 32
SQRT_D = math.sqrt(EMBED_DIM)

NC = 2
NS = 16
NW = NC * NS


_NORM_BLOCK = 2000


def _normalize_body(t_ref, o_ref):
    x = t_ref[...]
    ssq = jnp.sum(x * x, axis=-1, keepdims=True)
    norm = jnp.maximum(jnp.sqrt(ssq), 1e-12)
    o_ref[...] = x * (SQRT_D / norm)


def _normalize_table(table):
    n = table.shape[0]
    grid = n // _NORM_BLOCK
    return pl.pallas_call(
        _normalize_body,
        out_shape=jax.ShapeDtypeStruct(table.shape, table.dtype),
        grid=(grid,),
        in_specs=[pl.BlockSpec((_NORM_BLOCK, EMBED_DIM), lambda i: (i, 0))],
        out_specs=pl.BlockSpec((_NORM_BLOCK, EMBED_DIM), lambda i: (i, 0)),
    )(table)


_CHUNK = 1600


def _make_gather(batch, hist):
    total_rows = batch * hist
    b_per_w = total_rows // NW
    n_chunks = b_per_w // _CHUNK
    bat_per_chunk = _CHUNK // hist
    assert bat_per_chunk * hist == _CHUNK
    assert n_chunks >= 6 and n_chunks % 2 == 0
    mesh = plsc.VectorSubcoreMesh(
        core_axis_name="c", subcore_axis_name="s", num_cores=NC, num_subcores=NS
    )

    @functools.partial(
        pl.kernel,
        out_type=jax.ShapeDtypeStruct((batch, hist, EMBED_DIM), jnp.float32),
        mesh=mesh,
        scratch_types=[
            pltpu.VMEM((_CHUNK,), jnp.int32),
            pltpu.VMEM((_CHUNK,), jnp.int32),
            pltpu.VMEM((_CHUNK, EMBED_DIM), jnp.float32),
            pltpu.VMEM((_CHUNK, EMBED_DIM), jnp.float32),
            pltpu.SemaphoreType.DMA,
            pltpu.SemaphoreType.DMA,
            pltpu.SemaphoreType.DMA,
            pltpu.SemaphoreType.DMA,
            pltpu.SemaphoreType.DMA,
            pltpu.SemaphoreType.DMA,
        ],
        compiler_params=pltpu.CompilerParams(use_tc_tiling_on_sc=False),
    )
    def gather_k(idx_hbm, table_hbm, out_hbm, i0, i1, r0, r1, si0, si1, sg0, sg1, so0, so1):
        wid = lax.axis_index("s") * NC + lax.axis_index("c")
        base0 = wid * b_per_w
        I, R = (i0, i1), (r0, r1)
        SI, SG, SO = (si0, si1), (sg0, sg1), (so0, so1)

        def off(c):
            return pl.multiple_of(base0 + c * _CHUNK, 8)

        def idx_start(c, b):
            pltpu.async_copy(idx_hbm.at[pl.ds(off(c), _CHUNK)], I[b], SI[b])

        def idx_wait(b):
            pltpu.make_async_copy(
                idx_hbm.at[pl.ds(off(0), _CHUNK)], I[b], SI[b]
            ).wait()

        def gather_start(b):
            pltpu.async_copy(table_hbm.at[I[b]], R[b], SG[b])

        def gather_wait(b):
            pltpu.make_async_copy(table_hbm.at[I[b]], R[b], SG[b]).wait()

        def out_start(c, b):
            bbase = wid * (b_per_w // hist) + c * bat_per_chunk
            for j in range(bat_per_chunk):
                pltpu.async_copy(
                    R[b].at[pl.ds(j * hist, hist)], out_hbm.at[bbase + j], SO[b]
                )

        def out_wait(b):
            for j in range(bat_per_chunk):
                pltpu.make_async_copy(
                    R[b].at[pl.ds(j * hist, hist)], out_hbm.at[0], SO[b]
                ).wait()

        def step(c, b, prefetch=True, first=False):
            gather_wait(1 - b)
            out_start(c - 1, 1 - b)
            if prefetch:
                idx_start(c + 1, 1 - b)
            idx_wait(b)
            if not first:
                out_wait(b)
            gather_start(b)

        idx_start(0, 0)
        idx_wait(0)
        gather_start(0)
        idx_start(1, 1)
        step(1, 1, prefetch=True, first=True)

        def body(g, carry):
            c = 2 * g + 2
            step(c, 0)
            step(c + 1, 1)
            return carry

        lax.fori_loop(0, (n_chunks - 4) // 2, body, 0)

        step(n_chunks - 2, (n_chunks - 2) % 2, prefetch=True)
        step(n_chunks - 1, (n_chunks - 1) % 2, prefetch=False)
        bl = (n_chunks - 1) % 2
        gather_wait(bl)
        out_start(n_chunks - 1, bl)
        out_wait(1 - bl)
        out_wait(bl)

    return gather_k


def kernel(ids_or_weights, embedding_weight):
    table_n = _normalize_table(embedding_weight)
    batch, hist = ids_or_weights.shape
    idx = ids_or_weights.reshape(-1)
    return _make_gather(batch, hist)(idx, table_n)


# speedup vs baseline: 12.8348x; 2.0255x over previous
import functools
import math

import jax
import jax.numpy as jnp
from jax import lax
from jax.experimental import pallas as pl
from jax.experimental.pallas import tpu as pltpu
from jax.experimental.pallas import tpu_sc as plsc

NUM_CLASSES = 100000
EMBED_DIM = 32
SQRT_D = math.sqrt(EMBED_DIM)

NC = 2
NS = 16
NW = NC * NS


_NORM_BLOCK = 2000


def _normalize_body(t_ref, o_ref):
    x = t_ref[...]
    ssq = jnp.sum(x * x, axis=-1, keepdims=True)
    norm = jnp.maximum(jnp.sqrt(ssq), 1e-12)
    o_ref[...] = x * (SQRT_D / norm)


def _normalize_table(table):
    n = table.shape[0]
    grid = n // _NORM_BLOCK
    return pl.pallas_call(
        _normalize_body,
        out_shape=jax.ShapeDtypeStruct(table.shape, table.dtype),
        grid=(grid,),
        in_specs=[pl.BlockSpec((_NORM_BLOCK, EMBED_DIM), lambda i: (i, 0))],
        out_specs=pl.BlockSpec((_NORM_BLOCK, EMBED_DIM), lambda i: (i, 0)),
    )(table)


_CHUNK = 1600


def _make_gather(batch, hist):
    total_rows = batch * hist
    b_per_w = total_rows // NW
    n_chunks = b_per_w // _CHUNK
    bat_per_chunk = _CHUNK // hist
    assert bat_per_chunk * hist == _CHUNK
    assert n_chunks >= 6 and n_chunks % 2 == 0
    mesh = plsc.VectorSubcoreMesh(
        core_axis_name="c", subcore_axis_name="s", num_cores=NC, num_subcores=NS
    )

    @functools.partial(
        pl.kernel,
        out_type=jax.ShapeDtypeStruct((batch, hist, 128), jnp.float32),
        mesh=mesh,
        scratch_types=[
            pltpu.VMEM((_CHUNK,), jnp.int32),
            pltpu.VMEM((_CHUNK,), jnp.int32),
            pltpu.VMEM((_CHUNK, EMBED_DIM), jnp.float32),
            pltpu.VMEM((_CHUNK, EMBED_DIM), jnp.float32),
            pltpu.SemaphoreType.DMA,
            pltpu.SemaphoreType.DMA,
            pltpu.SemaphoreType.DMA,
            pltpu.SemaphoreType.DMA,
            pltpu.SemaphoreType.DMA,
            pltpu.SemaphoreType.DMA,
        ],
        compiler_params=pltpu.CompilerParams(use_tc_tiling_on_sc=False),
    )
    def gather_k(idx_hbm, table_hbm, out_hbm, i0, i1, r0, r1, si0, si1, sg0, sg1, so0, so1):
        wid = lax.axis_index("s") * NC + lax.axis_index("c")
        base0 = wid * b_per_w
        I, R = (i0, i1), (r0, r1)
        SI, SG, SO = (si0, si1), (sg0, sg1), (so0, so1)

        def off(c):
            return pl.multiple_of(base0 + c * _CHUNK, 8)

        def idx_start(c, b):
            pltpu.async_copy(idx_hbm.at[pl.ds(off(c), _CHUNK)], I[b], SI[b])

        def idx_wait(b):
            pltpu.make_async_copy(
                idx_hbm.at[pl.ds(off(0), _CHUNK)], I[b], SI[b]
            ).wait()

        def gather_start(b):
            pltpu.async_copy(table_hbm.at[I[b]], R[b], SG[b])

        def gather_wait(b):
            pltpu.make_async_copy(table_hbm.at[I[b]], R[b], SG[b]).wait()

        def out_start(c, b):
            bbase = wid * (b_per_w // hist) + c * bat_per_chunk
            for j in range(bat_per_chunk):
                pltpu.async_copy(
                    R[b].at[pl.ds(j * hist, hist)],
                    out_hbm.at[bbase + j, :, pl.ds(0, EMBED_DIM)],
                    SO[b],
                )

        def out_wait(b):
            for j in range(bat_per_chunk):
                pltpu.make_async_copy(
                    R[b].at[pl.ds(j * hist, hist)],
                    out_hbm.at[0, :, pl.ds(0, EMBED_DIM)],
                    SO[b],
                ).wait()

        def step(c, b, prefetch=True, first=False):
            gather_wait(1 - b)
            out_start(c - 1, 1 - b)
            if prefetch:
                idx_start(c + 1, 1 - b)
            idx_wait(b)
            if not first:
                out_wait(b)
            gather_start(b)

        idx_start(0, 0)
        idx_wait(0)
        gather_start(0)
        idx_start(1, 1)
        step(1, 1, prefetch=True, first=True)

        def body(g, carry):
            c = 2 * g + 2
            step(c, 0)
            step(c + 1, 1)
            return carry

        lax.fori_loop(0, (n_chunks - 4) // 2, body, 0)

        step(n_chunks - 2, (n_chunks - 2) % 2, prefetch=True)
        step(n_chunks - 1, (n_chunks - 1) % 2, prefetch=False)
        bl = (n_chunks - 1) % 2
        gather_wait(bl)
        out_start(n_chunks - 1, bl)
        out_wait(1 - bl)
        out_wait(bl)

    return gather_k


def kernel(ids_or_weights, embedding_weight):
    table_n = _normalize_table(embedding_weight)
    batch, hist = ids_or_weights.shape
    idx = ids_or_weights.reshape(-1)
    out = _make_gather(batch, hist)(idx, table_n)
    return out[:, :, :EMBED_DIM]
